# baseline (speedup 1.0000x reference)
import jax
import jax.numpy as jnp
import numpy as np
from jax.experimental import pallas as pl
from jax.experimental.pallas import tpu as pltpu

NUM_EXPERTS = 8
TOP_K = 2
ROUTER_TAU = 2.0
V_TH = 1.0
EPS = 1e-5

_TAUS = (1.5 + (4.0 - 1.5) * np.arange(NUM_EXPERTS) / (NUM_EXPERTS - 1)).astype(
    np.float32
)


def _moe_kernel(taus_ref,
                x_ref, wr_ref, brr_ref, grr_ref, betarr_ref,
                w1_hbm, g1_ref, b1_ref, beta1_ref,
                w2_hbm, g2_ref, b2_ref, beta2_ref, out_ref,
                v_ref, w1raw_ref, w1s_ref, thr_ref, sc2_ref, sh2_ref,
                w2v_ref, state_ref, sems, sem2):
    t = pl.program_id(0)
    h = pl.program_id(1)
    E = NUM_EXPERTS
    B = x_ref.shape[2]
    rsq = np.float32(1.0 / np.sqrt(1.0 + EPS))

    @pl.when(jnp.logical_and(t == 0, h == 0))
    def _build():
        v_ref[...] = jnp.zeros_like(v_ref)
        for e in range(E):
            state_ref[e] = 0
            scale2r = g2_ref[e] * rsq
            sh2r = b2_ref[e] * scale2r + beta2_ref[e]
            sc2_ref[e] = jnp.swapaxes(scale2r, 0, 1)
            sh2_ref[e] = jnp.swapaxes(sh2r, 0, 1)

    xt = x_ref[0, 0]
    vh = v_ref[h]
    v = vh + (xt - vh) / ROUTER_TAU
    s = (v - V_TH >= 0.0).astype(jnp.float32)
    v_ref[h] = v * (1.0 - s)
    m = jnp.sum(s, axis=-1) / x_ref.shape[4]
    logits = jax.lax.dot_general(m, wr_ref[...],
                                 (((1,), (1,)), ((), ())),
                                 preferred_element_type=jnp.float32)
    scale_r = grr_ref[0] * rsq
    shift_r = brr_ref[0] * scale_r + betarr_ref[0]
    logits = logits * scale_r[None, :] + shift_r[None, :]
    lmax = jnp.max(logits, axis=-1, keepdims=True)
    ex = jnp.exp(logits - lmax)
    probs = ex / jnp.sum(ex, axis=-1, keepdims=True)
    col = jax.lax.broadcasted_iota(jnp.int32, (B, E), 1)
    p1 = jnp.max(probs, axis=-1)
    i1 = jnp.min(jnp.where(probs == p1[:, None], col, E), axis=-1)
    probs2 = jnp.where(col == i1[:, None], -1.0, probs)
    p2 = jnp.max(probs2, axis=-1)
    i2 = jnp.min(jnp.where(probs2 == p2[:, None], col, E), axis=-1)
    wsum = p1 + p2
    wa = p1 / wsum
    wb = p2 / wsum

    for b in range(B):
        for e_vec in (i1, i2):
            e = e_vec[b]

            @pl.when(state_ref[e] == 0)
            def _start(e=e):
                pltpu.make_async_copy(w1_hbm.at[e], w1raw_ref.at[e],
                                      sems.at[e]).start()
                state_ref[e] = 1

    for e in range(E):
        @pl.when(state_ref[e] == 1)
        def _fold(e=e):
            pltpu.make_async_copy(w1_hbm.at[e], w1raw_ref.at[e],
                                  sems.at[e]).wait()
            scale1r = g1_ref[e] * rsq
            sh1r = b1_ref[e] * scale1r + beta1_ref[e]
            scale1 = jnp.swapaxes(scale1r, 0, 1)
            w1s_ref[e] = (w1raw_ref[e] * scale1).astype(jnp.bfloat16)
            thr_ref[e] = taus_ref[e] - jnp.swapaxes(sh1r, 0, 1)
            state_ref[e] = 2

    for b in range(B):
        xb = xt[b]
        base = 2.0 * xb
        branches = []
        for e_vec, w_vec in ((i1, wa), (i2, wb)):
            e = e_vec[b]
            tau = taus_ref[e]
            w = w_vec[b]
            s1 = (xb >= tau).astype(jnp.bfloat16)
            hraw = jnp.dot(w1s_ref[e], s1,
                           preferred_element_type=jnp.float32)
            thr = thr_ref[e]
            rmax = jnp.max(hraw, axis=1, keepdims=True)
            pred = jnp.max(rmax - thr) >= 0.0
            base = base + w * sh2_ref[e]
            branches.append((pred, hraw, thr, e, w * sc2_ref[e]))
        out_ref[0, 0, b] = base

        for pred, hraw, thr, e, wscale2 in branches:
            @pl.when(pred)
            def _conv2(hraw=hraw, thr=thr, e=e, wscale2=wscale2, b=b):
                cp = pltpu.make_async_copy(w2_hbm.at[e], w2v_ref, sem2)
                cp.start()
                s2 = (hraw - thr >= 0.0).astype(jnp.float32)
                cp.wait()
                o = jnp.dot(w2v_ref[...], s2,
                            preferred_element_type=jnp.float32)
                out_ref[0, 0, b] = out_ref[0, 0, b] + wscale2 * o


def kernel(x, Wr, br, gr, betar, W1, b1, g1, beta1, W2, b2, g2, beta2):
    T, B, C, H, W = x.shape
    HW = H * W
    E = NUM_EXPERTS
    HID = W1.shape[1]

    NH = 2
    B2 = B // NH
    x4 = x.reshape(T, NH, B2, C, HW)
    taus = jnp.asarray(_TAUS)

    def _res(shape):
        return pl.BlockSpec(shape, lambda t, h, ts: (0,) * len(shape))

    grid_spec = pltpu.PrefetchScalarGridSpec(
        num_scalar_prefetch=1,
        grid=(T, NH),
        in_specs=[
            pl.BlockSpec((1, 1, B2, C, HW),
                         lambda t, h, ts: (t, h, 0, 0, 0)),
            _res((E, C)),
            _res((1, E)),
            _res((1, E)),
            _res((1, E)),
            pl.BlockSpec(memory_space=pl.ANY),
            _res((E, 1, HID)),
            _res((E, 1, HID)),
            _res((E, 1, HID)),
            pl.BlockSpec(memory_space=pl.ANY),
            _res((E, 1, C)),
            _res((E, 1, C)),
            _res((E, 1, C)),
        ],
        out_specs=pl.BlockSpec((1, 1, B2, C, HW),
                               lambda t, h, ts: (t, h, 0, 0, 0)),
        scratch_shapes=[
            pltpu.VMEM((NH, B2, C, HW), jnp.float32),
            pltpu.VMEM((E, HID, C), jnp.float32),
            pltpu.VMEM((E, HID, C), jnp.bfloat16),
            pltpu.VMEM((E, HID, 1), jnp.float32),
            pltpu.VMEM((E, C, 1), jnp.float32),
            pltpu.VMEM((E, C, 1), jnp.float32),
            pltpu.VMEM((C, HID), jnp.float32),
            pltpu.SMEM((E,), jnp.int32),
            pltpu.SemaphoreType.DMA((E,)),
            pltpu.SemaphoreType.DMA,
        ],
    )

    out = pl.pallas_call(
        _moe_kernel,
        grid_spec=grid_spec,
        out_shape=jax.ShapeDtypeStruct((T, NH, B2, C, HW), jnp.float32),
        compiler_params=pltpu.CompilerParams(
            dimension_semantics=("arbitrary", "arbitrary"),
        ),
    )(taus, x4, Wr, br.reshape(1, E), gr.reshape(1, E), betar.reshape(1, E),
      W1,
      g1.reshape(E, 1, HID), b1.reshape(E, 1, HID), beta1.reshape(E, 1, HID),
      W2,
      g2.reshape(E, 1, C), b2.reshape(E, 1, C), beta2.reshape(E, 1, C))

    return out.reshape(T, B, C, H, W)


# speedup vs baseline: 1.2524x; 1.2524x over previous
import jax
import jax.numpy as jnp
import numpy as np
from jax.experimental import pallas as pl
from jax.experimental.pallas import tpu as pltpu

NUM_EXPERTS = 8
TOP_K = 2
ROUTER_TAU = 2.0
V_TH = 1.0
EPS = 1e-5

_TAUS = (1.5 + (4.0 - 1.5) * np.arange(NUM_EXPERTS) / (NUM_EXPERTS - 1)).astype(
    np.float32
)


def _moe_kernel(taus_ref,
                x_ref, wr_ref, brr_ref, grr_ref, betarr_ref,
                w1_hbm, g1_ref, b1_ref, beta1_ref,
                w2_hbm, g2_ref, b2_ref, beta2_ref, out_ref,
                v_ref, w1raw_ref, w1s_ref, thr_ref, sc2_ref, sh2_ref,
                w2v_ref, state_ref, sems, sem2):
    t = pl.program_id(0)
    E = NUM_EXPERTS
    B = x_ref.shape[1]
    rsq = np.float32(1.0 / np.sqrt(1.0 + EPS))

    @pl.when(t == 0)
    def _build():
        v_ref[...] = jnp.zeros_like(v_ref)
        for e in range(E):
            state_ref[e] = 0
            scale2r = g2_ref[e] * rsq
            sh2r = b2_ref[e] * scale2r + beta2_ref[e]
            sc2_ref[e] = jnp.swapaxes(scale2r, 0, 1)
            sh2_ref[e] = jnp.swapaxes(sh2r, 0, 1)

    xt = x_ref[0]
    v = v_ref[...] + (xt - v_ref[...]) / ROUTER_TAU
    s = (v - V_TH >= 0.0).astype(jnp.float32)
    v_ref[...] = v * (1.0 - s)
    m = jnp.sum(s, axis=-1) / x_ref.shape[3]
    logits = jax.lax.dot_general(m, wr_ref[...],
                                 (((1,), (1,)), ((), ())),
                                 preferred_element_type=jnp.float32)
    scale_r = grr_ref[0] * rsq
    shift_r = brr_ref[0] * scale_r + betarr_ref[0]
    logits = logits * scale_r[None, :] + shift_r[None, :]
    lmax = jnp.max(logits, axis=-1, keepdims=True)
    ex = jnp.exp(logits - lmax)
    probs = ex / jnp.sum(ex, axis=-1, keepdims=True)
    col = jax.lax.broadcasted_iota(jnp.int32, (B, E), 1)
    p1 = jnp.max(probs, axis=-1)
    i1 = jnp.min(jnp.where(probs == p1[:, None], col, E), axis=-1)
    probs2 = jnp.where(col == i1[:, None], -1.0, probs)
    p2 = jnp.max(probs2, axis=-1)
    i2 = jnp.min(jnp.where(probs2 == p2[:, None], col, E), axis=-1)
    wsum = p1 + p2
    wa = p1 / wsum
    wb = p2 / wsum

    for b in range(B):
        for e_vec in (i1, i2):
            e = e_vec[b]

            @pl.when(state_ref[e] == 0)
            def _start(e=e):
                pltpu.make_async_copy(w1_hbm.at[e], w1raw_ref.at[e],
                                      sems.at[e]).start()
                state_ref[e] = 1

    for e in range(E):
        @pl.when(state_ref[e] == 1)
        def _fold(e=e):
            pltpu.make_async_copy(w1_hbm.at[e], w1raw_ref.at[e],
                                  sems.at[e]).wait()
            scale1r = g1_ref[e] * rsq
            sh1r = b1_ref[e] * scale1r + beta1_ref[e]
            scale1 = jnp.swapaxes(scale1r, 0, 1)
            w1s_ref[e] = (w1raw_ref[e] * scale1).astype(jnp.bfloat16)
            thr_ref[e] = taus_ref[e] - jnp.swapaxes(sh1r, 0, 1)
            state_ref[e] = 2

    for b in range(B):
        xb = xt[b]
        base = 2.0 * xb
        branches = []
        for e_vec, w_vec in ((i1, wa), (i2, wb)):
            e = e_vec[b]
            tau = taus_ref[e]
            w = w_vec[b]
            s1 = (xb >= tau).astype(jnp.bfloat16)
            hraw = jnp.dot(w1s_ref[e], s1,
                           preferred_element_type=jnp.float32)
            thr = thr_ref[e]
            rmax = jnp.max(hraw, axis=1, keepdims=True)
            pred = jnp.max(rmax - thr) >= 0.0
            base = base + w * sh2_ref[e]
            branches.append((pred, hraw, thr, e, w * sc2_ref[e]))
        out_ref[0, b] = base

        for pred, hraw, thr, e, wscale2 in branches:
            @pl.when(pred)
            def _conv2(hraw=hraw, thr=thr, e=e, wscale2=wscale2, b=b):
                cp = pltpu.make_async_copy(w2_hbm.at[e], w2v_ref, sem2)
                cp.start()
                s2 = (hraw - thr >= 0.0).astype(jnp.float32)
                cp.wait()
                o = jnp.dot(w2v_ref[...], s2,
                            preferred_element_type=jnp.float32)
                out_ref[0, b] = out_ref[0, b] + wscale2 * o


def kernel(x, Wr, br, gr, betar, W1, b1, g1, beta1, W2, b2, g2, beta2):
    T, B, C, H, W = x.shape
    HW = H * W
    E = NUM_EXPERTS
    HID = W1.shape[1]

    x4 = x.reshape(T, B, C, HW)
    taus = jnp.asarray(_TAUS)

    def _res(shape):
        return pl.BlockSpec(shape, lambda t, ts: (0,) * len(shape))

    grid_spec = pltpu.PrefetchScalarGridSpec(
        num_scalar_prefetch=1,
        grid=(T,),
        in_specs=[
            pl.BlockSpec((1, B, C, HW), lambda t, ts: (t, 0, 0, 0)),
            _res((E, C)),
            _res((1, E)),
            _res((1, E)),
            _res((1, E)),
            pl.BlockSpec(memory_space=pl.ANY),
            _res((E, 1, HID)),
            _res((E, 1, HID)),
            _res((E, 1, HID)),
            pl.BlockSpec(memory_space=pl.ANY),
            _res((E, 1, C)),
            _res((E, 1, C)),
            _res((E, 1, C)),
        ],
        out_specs=pl.BlockSpec((1, B, C, HW), lambda t, ts: (t, 0, 0, 0)),
        scratch_shapes=[
            pltpu.VMEM((B, C, HW), jnp.float32),
            pltpu.VMEM((E, HID, C), jnp.float32),
            pltpu.VMEM((E, HID, C), jnp.bfloat16),
            pltpu.VMEM((E, HID, 1), jnp.float32),
            pltpu.VMEM((E, C, 1), jnp.float32),
            pltpu.VMEM((E, C, 1), jnp.float32),
            pltpu.VMEM((C, HID), jnp.float32),
            pltpu.SMEM((E,), jnp.int32),
            pltpu.SemaphoreType.DMA((E,)),
            pltpu.SemaphoreType.DMA,
        ],
    )

    out = pl.pallas_call(
        _moe_kernel,
        grid_spec=grid_spec,
        out_shape=jax.ShapeDtypeStruct((T, B, C, HW), jnp.float32),
        compiler_params=pltpu.CompilerParams(
            dimension_semantics=("arbitrary",),
        ),
    )(taus, x4, Wr, br.reshape(1, E), gr.reshape(1, E), betar.reshape(1, E),
      W1,
      g1.reshape(E, 1, HID), b1.reshape(E, 1, HID), beta1.reshape(E, 1, HID),
      W2,
      g2.reshape(E, 1, C), b2.reshape(E, 1, C), beta2.reshape(E, 1, C))

    return out.reshape(T, B, C, H, W)
